# baseline (speedup 1.0000x reference)
import functools

import jax
import jax.numpy as jnp
from jax import lax
from jax.experimental import pallas as pl
from jax.experimental.pallas import tpu as pltpu
from jax.experimental.pallas import tpu_sc as plsc

_B, _S, _D = 4, 8192, 2048
_NPOS = 64
_BS = _B * _S
_NC, _NS = 2, 16
_NW = _NC * _NS
_ROWS = _B * _NPOS
_RPW = _ROWS // _NW


_NCHUNK = 64
_CHUNK = _BS // _NCHUNK
_NBUF = 4
_LOOKAHEAD = 2


@functools.cache
def _tc_copy():
    @functools.partial(
        pl.kernel,
        mesh=pltpu.create_tensorcore_mesh("core"),
        scratch_types=[
            pltpu.VMEM((_NBUF, _CHUNK, _D), jnp.float32),
            [pltpu.SemaphoreType.DMA] * _NBUF,
            [pltpu.SemaphoreType.DMA] * _NBUF,
        ],
    )
    def body(in_hbm, out_hbm, buf, s_in, s_out):
        in_dma = [None] * _NCHUNK
        out_dma = [None] * _NBUF
        for k in range(_NCHUNK + _LOOKAHEAD):
            if k < _NCHUNK:
                slot = k % _NBUF
                if out_dma[slot] is not None:
                    out_dma[slot].wait()
                d = pltpu.make_async_copy(
                    in_hbm.at[pl.ds(k * _CHUNK, _CHUNK)], buf.at[slot],
                    s_in[slot])
                d.start()
                in_dma[k] = d
            if k >= _LOOKAHEAD:
                kk = k - _LOOKAHEAD
                slot = kk % _NBUF
                in_dma[kk].wait()
                o = pltpu.make_async_copy(
                    buf.at[slot], out_hbm.at[pl.ds(kk * _CHUNK, _CHUNK)],
                    s_out[slot])
                o.start()
                out_dma[slot] = o
        for o in out_dma:
            o.wait()

    return body


@functools.cache
def _sc_scatter():
    @functools.partial(
        pl.kernel,
        mesh=plsc.VectorSubcoreMesh(
            core_axis_name="c", subcore_axis_name="s",
            num_cores=_NC, num_subcores=_NS,
        ),
        scratch_types=[
            pltpu.VMEM((_RPW,), jnp.int32),
            pltpu.VMEM((_RPW, _D), jnp.float32),
            pltpu.SemaphoreType.DMA,
            pltpu.SemaphoreType.DMA,
        ],
    )
    def body(act_hbm, idx_hbm, out_hbm, idx_v, act_v, s_idx, s_act):
        w = lax.axis_index("s") * _NC + lax.axis_index("c")
        g = (w * _RPW) % _NPOS
        st_idx = pltpu.make_async_copy(idx_hbm.at[w], idx_v, s_idx)
        st_idx.start()
        st_act = pltpu.make_async_copy(act_hbm.at[pl.ds(g, _RPW)], act_v, s_act)
        st_act.start()
        st_idx.wait()
        st_act.wait()
        pltpu.async_copy(act_v, out_hbm.at[idx_v], s_idx).wait()

    return body


def kernel(output, activation, token_position):
    flat = output.reshape(_BS, _D)
    row_idx = (
        token_position[None, :].astype(jnp.int32)
        + (jnp.arange(_B, dtype=jnp.int32) * _S)[:, None]
    ).reshape(_NW, _RPW)
    out_ref = jax.new_ref(lax.empty((_BS, _D), jnp.float32))
    _tc_copy()(flat, out_ref)
    _sc_scatter()(activation, row_idx, out_ref)
    return jax.freeze(out_ref).reshape(_B, _S, _D)
